# Initial kernel scaffold; baseline (speedup 1.0000x reference)
import jax
import jax.numpy as jnp
from jax.experimental import pallas as pl


def kernel(x, edge_index, W1l, b1l, W1r, W2l, b2l, W2r):
    raise NotImplementedError("write your pallas kernel here")


# speedup vs baseline: 3.0827x; 3.0827x over previous
import jax
import jax.numpy as jnp
from jax import lax
from jax.experimental import pallas as pl
from jax.experimental.pallas import tpu as pltpu
from jax.experimental.pallas import tpu_sc as plsc

N = 10000
E = 160000
D = 256
HALF = 128
NC = 2
NS = 16
LANES = 16
CH = 128
SUBC = 16
SUP = 5
CHUNKS = SUP * SUBC
E_PAD = NS * CHUNKS * CH
CCH = E_PAD // (NC * NS * CH)
N_PAD = 10112
ROWS_PER_TILE = N_PAD // NS
BN = 400
GRID = N // BN

_MESH = dict(core_axis_name="c", subcore_axis_name="s")


def _zero_vmem(ref, rows, cols):
    zv = jnp.zeros((LANES,), jnp.float32)

    def zrow(i, _):
        def zcol(j, _):
            ref[i, pl.ds(j * LANES, LANES)] = zv
            return 0
        return lax.fori_loop(0, cols // LANES, zcol, 0)
    lax.fori_loop(0, rows, zrow, 0)


def _zero_shared_rows(shared, zsrc, base, rows, chunk_rows):
    full = rows // chunk_rows
    tail = rows % chunk_rows
    for k in range(full):
        pltpu.sync_copy(zsrc, shared.at[pl.ds(base + k * chunk_rows,
                                              chunk_rows)])
    if tail:
        pltpu.sync_copy(zsrc.at[pl.ds(0, tail)],
                        shared.at[pl.ds(base + full * chunk_rows, tail)])


def _sc_agg_body(x_hbm, src_hbm, dst_hbm, sum_hbm,
                 acc, src_b, dst_b, gbuf, gsem):
    c = lax.axis_index("c")
    s = lax.axis_index("s")

    _zero_vmem(gbuf, CH, HALF)
    base = s * ROWS_PER_TILE
    _zero_shared_rows(acc, gbuf, base, ROWS_PER_TILE, CH)

    offv = jnp.broadcast_to(c * N, (LANES,)).astype(jnp.int32)
    plsc.subcore_barrier()

    def sup(g, _):
        pltpu.sync_copy(src_hbm.at[s, g], src_b)
        pltpu.sync_copy(dst_hbm.at[s, g], dst_b)

        def orow(i, _):
            def ocol(j, _):
                sl = pl.ds(j * LANES, LANES)
                src_b[i, sl] = src_b[i, sl] + offv
                return 0
            return lax.fori_loop(0, CH // LANES, ocol, 0)
        lax.fori_loop(0, SUBC, orow, 0)

        def chunk(j, _):
            pltpu.async_copy(x_hbm.at[src_b.at[j]], gbuf, gsem).wait()
            pltpu.sync_copy(gbuf, acc.at[dst_b.at[j]], add=True)
            return 0
        lax.fori_loop(0, SUBC, chunk, 0)
        return 0
    lax.fori_loop(0, SUP, sup, 0)

    plsc.subcore_barrier()
    pltpu.sync_copy(acc.at[pl.ds(base, ROWS_PER_TILE)],
                    sum_hbm.at[pl.ds(c * N_PAD + base, ROWS_PER_TILE)])


_sc_agg = pl.kernel(
    _sc_agg_body,
    out_type=jax.ShapeDtypeStruct((NC * N_PAD, HALF), jnp.float32),
    mesh=plsc.VectorSubcoreMesh(**_MESH),
    scratch_types=[
        pltpu.VMEM_SHARED((N_PAD, HALF), jnp.float32),
        pltpu.VMEM((SUBC, CH), jnp.int32),
        pltpu.VMEM((SUBC, CH), jnp.int32),
        pltpu.VMEM((CH, HALF), jnp.float32),
        pltpu.SemaphoreType.DMA,
    ],
)


def _sc_counts_body(dst_hbm, cnt_hbm, cacc, dst_v, zb, ones_v):
    c = lax.axis_index("c")
    s = lax.axis_index("s")

    pltpu.sync_copy(dst_hbm.at[c, s], dst_v)
    _zero_vmem(zb, CH, HALF)
    ov = jnp.full((LANES,), 1.0, jnp.float32)

    def orow(i, _):
        def ocol(j, _):
            ones_v[i, pl.ds(j * LANES, LANES)] = ov
            return 0
        return lax.fori_loop(0, HALF // LANES, ocol, 0)
    lax.fori_loop(0, CH, orow, 0)

    base = s * ROWS_PER_TILE
    _zero_shared_rows(cacc, zb, base, ROWS_PER_TILE, CH)
    plsc.subcore_barrier()

    def chunk(j, _):
        pltpu.sync_copy(ones_v, cacc.at[dst_v.at[j]], add=True)
        return 0
    lax.fori_loop(0, CCH, chunk, 0)

    plsc.subcore_barrier()
    pltpu.sync_copy(cacc.at[pl.ds(base, ROWS_PER_TILE)],
                    cnt_hbm.at[pl.ds(c * N_PAD + base, ROWS_PER_TILE)])


_sc_counts = pl.kernel(
    _sc_counts_body,
    out_type=jax.ShapeDtypeStruct((NC * N_PAD, HALF), jnp.float32),
    mesh=plsc.VectorSubcoreMesh(**_MESH),
    scratch_types=[
        pltpu.VMEM_SHARED((N_PAD, HALF), jnp.float32),
        pltpu.VMEM((CCH, CH), jnp.int32),
        pltpu.VMEM((CH, HALF), jnp.float32),
        pltpu.VMEM((CH, HALF), jnp.float32),
    ],
)


def _dense_body(relu_split):
    def body(s_ref, x_ref, cnt_ref, wl_ref, wr_ref, b_ref, *outs):
        cnt = cnt_ref[0][:, 0:1] + cnt_ref[1][:, 0:1]
        inv = 1.0 / jnp.maximum(cnt, 1.0)
        m = jnp.concatenate([s_ref[0], s_ref[1]], axis=1) * inv
        acc = jnp.dot(m, wl_ref[...], preferred_element_type=jnp.float32)
        acc = acc + jnp.dot(x_ref[...], wr_ref[...],
                            preferred_element_type=jnp.float32)
        acc = acc + b_ref[...]
        if relu_split:
            acc = jnp.maximum(acc, 0.0)
            outs[0][0] = acc[:, :HALF]
            outs[0][1] = acc[:, HALF:]
            outs[1][...] = acc
        else:
            outs[0][...] = acc
    return body


def _dense_call(s, xin, cnt, wlT, wrT, b, relu_split):
    in_specs = [
        pl.BlockSpec((NC, BN, HALF), lambda i: (0, i, 0)),
        pl.BlockSpec((BN, D), lambda i: (i, 0)),
        pl.BlockSpec((NC, BN, HALF), lambda i: (0, i, 0)),
        pl.BlockSpec((D, D), lambda i: (0, 0)),
        pl.BlockSpec((D, D), lambda i: (0, 0)),
        pl.BlockSpec((1, D), lambda i: (0, 0)),
    ]
    if relu_split:
        out_shape = [jax.ShapeDtypeStruct((NC, N, HALF), jnp.float32),
                     jax.ShapeDtypeStruct((N, D), jnp.float32)]
        out_specs = [pl.BlockSpec((NC, BN, HALF), lambda i: (0, i, 0)),
                     pl.BlockSpec((BN, D), lambda i: (i, 0))]
    else:
        out_shape = jax.ShapeDtypeStruct((N, D), jnp.float32)
        out_specs = pl.BlockSpec((BN, D), lambda i: (i, 0))
    return pl.pallas_call(
        _dense_body(relu_split),
        grid=(GRID,),
        in_specs=in_specs,
        out_specs=out_specs,
        out_shape=out_shape,
    )(s, xin, cnt, wlT, wrT, b)


def kernel(x, edge_index, W1l, b1l, W1r, W2l, b2l, W2r):
    src = edge_index[0].astype(jnp.int32)
    dst = edge_index[1].astype(jnp.int32)
    padn = E_PAD - E
    src_p = jnp.concatenate(
        [src, jnp.zeros((padn,), jnp.int32)]).reshape(NS, SUP, SUBC, CH)
    dst_flat = jnp.concatenate([dst, jnp.full((padn,), N, jnp.int32)])
    dst_p = dst_flat.reshape(NS, SUP, SUBC, CH)
    dst_c = dst_flat.reshape(NC, NS, CCH, CH)

    x_flat = x.reshape(N, NC, HALF).transpose(1, 0, 2).reshape(NC * N, HALF)

    cnt = _sc_counts(dst_c).reshape(NC, N_PAD, HALF)
    sum1 = _sc_agg(x_flat, src_p, dst_p).reshape(NC, N_PAD, HALF)
    h_split, h_full = _dense_call(sum1, x, cnt, W1l.T, W1r.T,
                                  b1l.reshape(1, D), True)

    h_flat = h_split.reshape(NC * N, HALF)
    sum2 = _sc_agg(h_flat, src_p, dst_p).reshape(NC, N_PAD, HALF)
    out = _dense_call(sum2, h_full, cnt, W2l.T, W2r.T,
                      b2l.reshape(1, D), False)
    return out


# speedup vs baseline: 3.5933x; 1.1657x over previous
import jax
import jax.numpy as jnp
from jax import lax
from jax.experimental import pallas as pl
from jax.experimental.pallas import tpu as pltpu
from jax.experimental.pallas import tpu_sc as plsc

N = 10000
E = 160000
D = 256
HALF = 128
NC = 2
NS = 16
LANES = 16
CH = 128
SUBC = 8
SUP = 10
CHUNKS = SUP * SUBC
E_PAD = NS * CHUNKS * CH
CCH = E_PAD // (NC * NS * CH)
N_PAD = 10112
ROWS_PER_TILE = N_PAD // NS
BN = 400
GRID = N // BN

_MESH = dict(core_axis_name="c", subcore_axis_name="s")


def _zero_vmem(ref, rows, cols):
    zv = jnp.zeros((LANES,), jnp.float32)

    def zrow(i, _):
        def zcol(j, _):
            ref[i, pl.ds(j * LANES, LANES)] = zv
            return 0
        return lax.fori_loop(0, cols // LANES, zcol, 0)
    lax.fori_loop(0, rows, zrow, 0)


def _zero_shared_rows(shared, zsrc, base, rows, chunk_rows):
    full = rows // chunk_rows
    tail = rows % chunk_rows
    for k in range(full):
        pltpu.sync_copy(zsrc, shared.at[pl.ds(base + k * chunk_rows,
                                              chunk_rows)])
    if tail:
        pltpu.sync_copy(zsrc.at[pl.ds(0, tail)],
                        shared.at[pl.ds(base + full * chunk_rows, tail)])


def _sc_agg_body(x_hbm, src_hbm, dst_hbm, sum_hbm,
                 acc, src_b, dst_b, gb0, gb1, gs0, gs1, ss0, ss1):
    c = lax.axis_index("c")
    s = lax.axis_index("s")
    gb = (gb0, gb1)
    gs = (gs0, gs1)
    ss = (ss0, ss1)

    _zero_vmem(gb0, CH, HALF)
    base = s * ROWS_PER_TILE
    _zero_shared_rows(acc, gb0, base, ROWS_PER_TILE, CH)

    offv = jnp.broadcast_to(c * N, (LANES,)).astype(jnp.int32)
    plsc.subcore_barrier()

    def sup(g, _):
        pltpu.sync_copy(src_hbm.at[s * SUP + g], src_b)
        pltpu.sync_copy(dst_hbm.at[s * SUP + g], dst_b)

        def orow(i, _):
            def ocol(j, _):
                sl = pl.ds(j * LANES, LANES)
                src_b[i, sl] = src_b[i, sl] + offv
                return 0
            return lax.fori_loop(0, CH // LANES, ocol, 0)
        lax.fori_loop(0, SUBC, orow, 0)

        gathers = [None, None]
        scatters = [None, None]
        gathers[0] = pltpu.async_copy(x_hbm.at[src_b.at[0]], gb[0], gs[0])
        for j in range(SUBC):
            b = j % 2
            nb = (j + 1) % 2
            if j + 1 < SUBC:
                if scatters[nb] is not None:
                    scatters[nb].wait()
                    scatters[nb] = None
                gathers[nb] = pltpu.async_copy(
                    x_hbm.at[src_b.at[j + 1]], gb[nb], gs[nb])
            gathers[b].wait()
            scatters[b] = pltpu.async_copy(
                gb[b], acc.at[dst_b.at[j]], ss[b], add=True)
        for b in range(2):
            if scatters[b] is not None:
                scatters[b].wait()
        return 0
    lax.fori_loop(0, SUP, sup, 0)

    plsc.subcore_barrier()
    pltpu.sync_copy(acc.at[pl.ds(base, ROWS_PER_TILE)],
                    sum_hbm.at[pl.ds(c * N_PAD + base, ROWS_PER_TILE)])


_sc_agg = pl.kernel(
    _sc_agg_body,
    out_type=jax.ShapeDtypeStruct((NC * N_PAD, HALF), jnp.float32),
    mesh=plsc.VectorSubcoreMesh(**_MESH),
    scratch_types=[
        pltpu.VMEM_SHARED((N_PAD, HALF), jnp.float32),
        pltpu.VMEM((SUBC, CH), jnp.int32),
        pltpu.VMEM((SUBC, CH), jnp.int32),
        pltpu.VMEM((CH, HALF), jnp.float32),
        pltpu.VMEM((CH, HALF), jnp.float32),
        pltpu.SemaphoreType.DMA,
        pltpu.SemaphoreType.DMA,
        pltpu.SemaphoreType.DMA,
        pltpu.SemaphoreType.DMA,
    ],
)


def _sc_counts_body(dst_hbm, cnt_hbm, cacc, dst_v, zb, ones_v):
    c = lax.axis_index("c")
    s = lax.axis_index("s")

    pltpu.sync_copy(dst_hbm.at[c, s], dst_v)
    _zero_vmem(zb, CH, HALF)
    ov = jnp.full((LANES,), 1.0, jnp.float32)

    def orow(i, _):
        def ocol(j, _):
            ones_v[i, pl.ds(j * LANES, LANES)] = ov
            return 0
        return lax.fori_loop(0, HALF // LANES, ocol, 0)
    lax.fori_loop(0, CH, orow, 0)

    base = s * ROWS_PER_TILE
    _zero_shared_rows(cacc, zb, base, ROWS_PER_TILE, CH)
    plsc.subcore_barrier()

    def chunk(j, _):
        pltpu.sync_copy(ones_v, cacc.at[dst_v.at[j]], add=True)
        return 0
    lax.fori_loop(0, CCH, chunk, 0)

    plsc.subcore_barrier()
    pltpu.sync_copy(cacc.at[pl.ds(base, ROWS_PER_TILE)],
                    cnt_hbm.at[pl.ds(c * N_PAD + base, ROWS_PER_TILE)])


_sc_counts = pl.kernel(
    _sc_counts_body,
    out_type=jax.ShapeDtypeStruct((NC * N_PAD, HALF), jnp.float32),
    mesh=plsc.VectorSubcoreMesh(**_MESH),
    scratch_types=[
        pltpu.VMEM_SHARED((N_PAD, HALF), jnp.float32),
        pltpu.VMEM((CCH, CH), jnp.int32),
        pltpu.VMEM((CH, HALF), jnp.float32),
        pltpu.VMEM((CH, HALF), jnp.float32),
    ],
)


def _dense_body(relu_split):
    def body(s_ref, x_ref, cnt_ref, wl_ref, wr_ref, b_ref, *outs):
        cnt = cnt_ref[0][:, 0:1] + cnt_ref[1][:, 0:1]
        inv = 1.0 / jnp.maximum(cnt, 1.0)
        m = jnp.concatenate([s_ref[0], s_ref[1]], axis=1) * inv
        acc = jnp.dot(m, wl_ref[...], preferred_element_type=jnp.float32)
        acc = acc + jnp.dot(x_ref[...], wr_ref[...],
                            preferred_element_type=jnp.float32)
        acc = acc + b_ref[...]
        if relu_split:
            acc = jnp.maximum(acc, 0.0)
            outs[0][0] = acc[:, :HALF]
            outs[0][1] = acc[:, HALF:]
            outs[1][...] = acc
        else:
            outs[0][...] = acc
    return body


def _dense_call(s, xin, cnt, wlT, wrT, b, relu_split):
    in_specs = [
        pl.BlockSpec((NC, BN, HALF), lambda i: (0, i, 0)),
        pl.BlockSpec((BN, D), lambda i: (i, 0)),
        pl.BlockSpec((NC, BN, 8), lambda i: (0, i, 0)),
        pl.BlockSpec((D, D), lambda i: (0, 0)),
        pl.BlockSpec((D, D), lambda i: (0, 0)),
        pl.BlockSpec((1, D), lambda i: (0, 0)),
    ]
    if relu_split:
        out_shape = [jax.ShapeDtypeStruct((NC, N, HALF), jnp.float32),
                     jax.ShapeDtypeStruct((N, D), jnp.float32)]
        out_specs = [pl.BlockSpec((NC, BN, HALF), lambda i: (0, i, 0)),
                     pl.BlockSpec((BN, D), lambda i: (i, 0))]
    else:
        out_shape = jax.ShapeDtypeStruct((N, D), jnp.float32)
        out_specs = pl.BlockSpec((BN, D), lambda i: (i, 0))
    return pl.pallas_call(
        _dense_body(relu_split),
        grid=(GRID,),
        in_specs=in_specs,
        out_specs=out_specs,
        out_shape=out_shape,
    )(s, xin, cnt, wlT, wrT, b)


def kernel(x, edge_index, W1l, b1l, W1r, W2l, b2l, W2r):
    src = edge_index[0].astype(jnp.int32)
    dst = edge_index[1].astype(jnp.int32)
    padn = E_PAD - E
    src_p = jnp.concatenate(
        [src, jnp.zeros((padn,), jnp.int32)]).reshape(NS * SUP, SUBC, CH)
    dst_flat = jnp.concatenate([dst, jnp.full((padn,), N, jnp.int32)])
    dst_p = dst_flat.reshape(NS * SUP, SUBC, CH)
    dst_c = dst_flat.reshape(NC, NS, CCH, CH)

    x_flat = x.reshape(N, NC, HALF).transpose(1, 0, 2).reshape(NC * N, HALF)

    cnt = _sc_counts(dst_c).reshape(NC, N_PAD, HALF)[:, :, :8]
    sum1 = _sc_agg(x_flat, src_p, dst_p).reshape(NC, N_PAD, HALF)
    h_split, h_full = _dense_call(sum1, x, cnt, W1l.T, W1r.T,
                                  b1l.reshape(1, D), True)

    h_flat = h_split.reshape(NC * N, HALF)
    sum2 = _sc_agg(h_flat, src_p, dst_p).reshape(NC, N_PAD, HALF)
    out = _dense_call(sum2, h_full, cnt, W2l.T, W2r.T,
                      b2l.reshape(1, D), False)
    return out
